# Initial kernel scaffold; baseline (speedup 1.0000x reference)
import jax
import jax.numpy as jnp
from jax.experimental import pallas as pl


def kernel(x, edge_index, W_l1, b_l1, W_r1, b_r1, att1, bias1, W_l2, b_l2, W_r2, b_r2, att2, bias2, W_cls, b_cls):
    raise NotImplementedError("write your pallas kernel here")


# speedup vs baseline: 3.2074x; 3.2074x over previous
import functools

import jax
import jax.numpy as jnp
from jax import lax
from jax.experimental import pallas as pl
from jax.experimental.pallas import tpu as pltpu
from jax.experimental.pallas import tpu_sc as plsc

_N = 10000
_E = 320000
_NC = 2
_NS = 16
_L = 16
_B = 80
_CW = 128
_PAD = 16
_CWP = _CW + _PAD
_EPS = 1e-16
_NEG = 0.2


def _mesh():
    return plsc.VectorSubcoreMesh(core_axis_name="c", subcore_axis_name="s")


def _lanesum(v, rbuf):
    for sh in (8, 4, 2, 1):
        rbuf[pl.ds(0, _L)] = v
        rbuf[pl.ds(_L, _L)] = v
        v = v + rbuf[pl.ds(sh, _L)]
    return v


def _make_alpha(nchunks):
    ew = _E // (_NC * _NS)
    nblk = ew // _B
    ngrp = _B // _L

    @functools.partial(
        pl.kernel,
        mesh=_mesh(),
        out_type=jax.ShapeDtypeStruct((_E,), jnp.float32),
        scratch_types=[
            pltpu.VMEM((nchunks, _B), jnp.int32),
            pltpu.VMEM((nchunks, _B), jnp.int32),
            pltpu.VMEM((nchunks * _B, _CW), jnp.float32),
            pltpu.VMEM((nchunks * _B, _CW), jnp.float32),
            pltpu.VMEM((_B,), jnp.float32),
            pltpu.VMEM((nchunks * _CW,), jnp.float32),
            pltpu.VMEM((2 * _L,), jnp.float32),
            pltpu.SemaphoreType.DMA,
        ],
        compiler_params=pltpu.CompilerParams(needs_layout_passes=False),
    )
    def alpha_kernel(src_h, dst_h, att_h, xls_h, xrs_h, ex_h,
                     sidx, didx, xlb, xrb, exb, attv, rbuf, sem):
        wid = lax.axis_index("s") * _NC + lax.axis_index("c")
        pltpu.sync_copy(att_h, attv)
        iot = lax.iota(jnp.int32, _L)
        attc = [[attv[pl.ds(c * _CW + k * _L, _L)] for k in range(_CW // _L)]
                for c in range(nchunks)]

        def blk_body(blk, carry):
            base = wid * ew + blk * _B
            pltpu.sync_copy(src_h.at[pl.ds(base, _B)], sidx.at[0])
            pltpu.sync_copy(dst_h.at[pl.ds(base, _B)], didx.at[0])
            for c in range(1, nchunks):
                for g in range(ngrp):
                    sl = pl.ds(g * _L, _L)
                    sidx[c, sl] = sidx[0, sl] + c * _N
                    didx[c, sl] = didx[0, sl] + c * _N
            descs = []
            for c in range(nchunks):
                dsl = pl.ds(c * _B, _B)
                descs.append(pltpu.async_copy(xls_h.at[sidx.at[c]], xlb.at[dsl], sem))
                descs.append(pltpu.async_copy(xrs_h.at[didx.at[c]], xrb.at[dsl], sem))
            for d in descs:
                d.wait()
            def grp_body(g2, carry2):
                av = jnp.zeros((_L,), jnp.float32)
                for lane in range(_L):
                    e2 = g2 * _L + lane
                    accv = jnp.zeros((_L,), jnp.float32)
                    for c in range(nchunks):
                        row = c * _B + e2
                        for k in range(_CW // _L):
                            sl = pl.ds(k * _L, _L)
                            z = xlb[row, sl] + xrb[row, sl]
                            z = jnp.maximum(z, _NEG * z)
                            accv = accv + attc[c][k] * z
                    accv = _lanesum(accv, rbuf)
                    av = av + jnp.where(iot == lane, accv, 0.0)
                exb[pl.ds(g2 * _L, _L)] = jnp.exp(av)
                return carry2

            lax.fori_loop(0, ngrp, grp_body, 0)
            pltpu.sync_copy(exb, ex_h.at[pl.ds(base, _B)])
            return carry

        lax.fori_loop(0, nblk, blk_body, 0)

    return alpha_kernel


def _make_passb(npass, col_split):
    if col_split:
        ew = _E // _NS
    else:
        ew = _E // (_NC * _NS)
    nblk = ew // _B
    ngrp = _B // _L
    rpt = 640
    rstride = 624
    zr = 40
    nchunks_out = npass * _NC if col_split else _NC
    nden = _N if col_split else _NC * _N

    @functools.partial(
        pl.kernel,
        mesh=_mesh(),
        out_type=[
            jax.ShapeDtypeStruct((nchunks_out * _N, _CW), jnp.float32),
            jax.ShapeDtypeStruct((nden, _CW), jnp.float32),
        ],
        scratch_types=[
            pltpu.VMEM((_B,), jnp.int32),
            pltpu.VMEM((_B,), jnp.int32),
            pltpu.VMEM((_B,), jnp.float32),
            pltpu.VMEM((_B, _CW), jnp.float32),
            pltpu.VMEM((_B, _CW), jnp.float32),
            pltpu.VMEM((zr, _CW), jnp.float32),
            pltpu.VMEM((rpt,), jnp.float32),
            pltpu.VMEM((_L, _CW), jnp.float32),
            pltpu.VMEM_SHARED((_N, _CW), jnp.float32),
            pltpu.VMEM_SHARED((_N,), jnp.float32),
            pltpu.SemaphoreType.DMA,
        ],
        compiler_params=pltpu.CompilerParams(needs_layout_passes=False),
    )
    def passb_kernel(src_h, dst_h, ex_h, xls_h, out_h, den_h,
                     sidx, didx, exb, xlb, vbuf, zbuf, dden, dbb,
                     accsp, denslab, sem):
        core = lax.axis_index("c")
        s = lax.axis_index("s")
        rowbase = s * rstride
        zv = jnp.zeros((_L,), jnp.float32)

        def zrow(r, carry):
            for k in range(_CW // _L):
                zbuf[r, pl.ds(k * _L, _L)] = zv
            return carry

        lax.fori_loop(0, zr, zrow, 0)

        def zden(i, carry):
            dden[pl.ds(i * _L, _L)] = zv
            return carry

        lax.fori_loop(0, rpt // _L, zden, 0)
        pltpu.sync_copy(dden, denslab.at[pl.ds(rowbase, rpt)])

        for p in range(npass):
            if col_split:
                chunk = core * npass + p
            else:
                chunk = core * 0
            rowoff = chunk * _N

            def zb(i, carry):
                pltpu.sync_copy(zbuf, accsp.at[pl.ds(rowbase + i * zr, zr)])
                return carry

            lax.fori_loop(0, rpt // zr, zb, 0)
            plsc.subcore_barrier()

            def blk_body(blk, carry, rowoff=rowoff, p=p):
                if col_split:
                    base = s * ew + blk * _B
                else:
                    base = core * (_E // _NC) + s * ew + blk * _B
                pltpu.sync_copy(src_h.at[pl.ds(base, _B)], sidx)
                pltpu.sync_copy(dst_h.at[pl.ds(base, _B)], didx)
                pltpu.sync_copy(ex_h.at[pl.ds(base, _B)], exb)
                if col_split:
                    for g in range(ngrp):
                        sl = pl.ds(g * _L, _L)
                        sidx[sl] = sidx[sl] + rowoff
                pltpu.async_copy(xls_h.at[sidx], xlb, sem).wait()

                def ebody(g2, carry2):
                    exv = exb[pl.ds(g2 * _L, _L)]
                    for lane in range(_L):
                        e2 = g2 * _L + lane
                        exs = exv[lane]
                        for k in range(_CW // _L):
                            sl = pl.ds(k * _L, _L)
                            vbuf[e2, sl] = xlb[e2, sl] * exs
                    return carry2

                lax.fori_loop(0, ngrp, ebody, 0)
                if p == 0:
                    pltpu.sync_copy(exb, denslab.at[didx], add=True)
                pltpu.sync_copy(vbuf, accsp.at[didx], add=True)
                return carry

            lax.fori_loop(0, nblk, blk_body, 0)
            plsc.subcore_barrier()

            outrow = rowoff + rowbase if col_split else core * _N + rowbase
            pltpu.sync_copy(accsp.at[pl.ds(rowbase, rpt)],
                            out_h.at[pl.ds(outrow, rpt)])

            if p == 0:
                def den_stage():
                    pltpu.sync_copy(denslab.at[pl.ds(rowbase, rpt)], dden)
                    denrow = rowbase if col_split else core * _N + rowbase

                    def dbc(g, carry):
                        dv = dden[pl.ds(g * _L, _L)]
                        for lane in range(_L):
                            bc = zv + dv[lane]
                            for k in range(_CW // _L):
                                dbb[lane, pl.ds(k * _L, _L)] = bc
                        pltpu.sync_copy(
                            dbb, den_h.at[pl.ds(denrow + g * _L, _L)])
                        return carry

                    lax.fori_loop(0, rpt // _L, dbc, 0)

                if col_split:
                    @pl.when(core == 0)
                    def _():
                        den_stage()
                else:
                    den_stage()
            plsc.subcore_barrier()

    return passb_kernel


def _transform1(x, W_l, b_l, W_r, b_r):
    nb = 1000
    grid = (4, _N // nb)

    def body(x_ref, wl_ref, bl_ref, wr_ref, br_ref, xl_ref, xr_ref):
        c = pl.program_id(0)
        xb = x_ref[...]
        bl = bl_ref[pl.ds(c, 1), :]
        br = br_ref[pl.ds(c, 1), :]
        xl_ref[...] = jnp.dot(xb, wl_ref[...],
                              preferred_element_type=jnp.float32) + bl
        xr_ref[...] = jnp.dot(xb, wr_ref[...],
                              preferred_element_type=jnp.float32) + br

    out = pl.pallas_call(
        body,
        grid=grid,
        in_specs=[
            pl.BlockSpec((nb, 128), lambda c, i: (i, 0)),
            pl.BlockSpec((128, 128), lambda c, i: (0, c)),
            pl.BlockSpec((4, 128), lambda c, i: (0, 0)),
            pl.BlockSpec((128, 128), lambda c, i: (0, c)),
            pl.BlockSpec((4, 128), lambda c, i: (0, 0)),
        ],
        out_specs=[
            pl.BlockSpec((nb, 128), lambda c, i: (c * grid[1] + i, 0)),
            pl.BlockSpec((nb, 128), lambda c, i: (c * grid[1] + i, 0)),
        ],
        out_shape=[
            jax.ShapeDtypeStruct((4 * _N, 128), jnp.float32),
            jax.ShapeDtypeStruct((4 * _N, 128), jnp.float32),
        ],
    )(x, W_l, b_l, W_r, b_r)
    return out


def _transform2(o1, den1, bias1, W_l2, b_l2, W_r2, b_r2):
    nb = 1000
    grid = (_N // nb,)

    def body(o0, o1c, o2c, o3c, d_ref, b1_ref, wl_ref, bl_ref, wr_ref, br_ref,
             xl_ref, xr_ref):
        den = d_ref[...] + _EPS
        acc_l = jnp.zeros((nb, 128), jnp.float32)
        acc_r = jnp.zeros((nb, 128), jnp.float32)
        for c, oc in enumerate((o0, o1c, o2c, o3c)):
            h = jnp.maximum(oc[...] / den + b1_ref[c:c + 1, :], 0.0)
            acc_l += jnp.dot(h, wl_ref[c * 128:(c + 1) * 128, :],
                             preferred_element_type=jnp.float32)
            acc_r += jnp.dot(h, wr_ref[c * 128:(c + 1) * 128, :],
                             preferred_element_type=jnp.float32)
        xl_ref[...] = acc_l + bl_ref[...]
        xr_ref[...] = acc_r + br_ref[...]

    g1 = grid[0]
    in_specs = [pl.BlockSpec((nb, 128), lambda i, c=c: (c * g1 + i, 0))
                for c in range(4)]
    in_specs += [
        pl.BlockSpec((nb, 128), lambda i: (i, 0)),
        pl.BlockSpec((4, 128), lambda i: (0, 0)),
        pl.BlockSpec((512, 128), lambda i: (0, 0)),
        pl.BlockSpec((1, 128), lambda i: (0, 0)),
        pl.BlockSpec((512, 128), lambda i: (0, 0)),
        pl.BlockSpec((1, 128), lambda i: (0, 0)),
    ]
    return pl.pallas_call(
        body,
        grid=grid,
        in_specs=in_specs,
        out_specs=[
            pl.BlockSpec((nb, 128), lambda i: (i, 0)),
            pl.BlockSpec((nb, 128), lambda i: (i, 0)),
        ],
        out_shape=[
            jax.ShapeDtypeStruct((_N, 128), jnp.float32),
            jax.ShapeDtypeStruct((_N, 128), jnp.float32),
        ],
    )(o1, o1, o1, o1, den1, bias1, W_l2, b_l2, W_r2, b_r2)


def _final(o2, den2, bias2, W_cls, b_cls):
    nb = 1000
    grid = (_N // nb,)
    g1 = grid[0]

    def body(p0, p1, d0, d1, b2_ref, wc_ref, bc_ref, out_ref, acc):
        i = pl.program_id(0)
        num = p0[...] + p1[...]
        den = d0[...] + d1[...] + _EPS
        h = jnp.maximum(num / den + b2_ref[...], 0.0)
        psum = jnp.sum(h, axis=0, keepdims=True)

        @pl.when(i == 0)
        def _():
            acc[...] = psum

        @pl.when(i > 0)
        def _():
            acc[...] = acc[...] + psum

        @pl.when(i == g1 - 1)
        def _():
            pooled = acc[...] / float(_N)
            logits = jnp.dot(pooled, wc_ref[...],
                             preferred_element_type=jnp.float32) + bc_ref[...]
            out_ref[...] = jax.nn.sigmoid(logits)

    return pl.pallas_call(
        body,
        grid=grid,
        in_specs=[
            pl.BlockSpec((nb, 128), lambda i: (i, 0)),
            pl.BlockSpec((nb, 128), lambda i: (g1 + i, 0)),
            pl.BlockSpec((nb, 128), lambda i: (i, 0)),
            pl.BlockSpec((nb, 128), lambda i: (g1 + i, 0)),
            pl.BlockSpec((1, 128), lambda i: (0, 0)),
            pl.BlockSpec((128, 10), lambda i: (0, 0)),
            pl.BlockSpec((1, 10), lambda i: (0, 0)),
        ],
        out_specs=pl.BlockSpec((1, 10), lambda i: (0, 0)),
        out_shape=jax.ShapeDtypeStruct((1, 10), jnp.float32),
        scratch_shapes=[pltpu.VMEM((1, 128), jnp.float32)],
    )(o2, o2, den2, den2, bias2, W_cls, b_cls)


_alpha1 = _make_alpha(4)
_alpha2 = _make_alpha(1)
_passb1 = _make_passb(2, True)
_passb2 = _make_passb(1, False)


def kernel(x, edge_index, W_l1, b_l1, W_r1, b_r1, att1, bias1,
           W_l2, b_l2, W_r2, b_r2, att2, bias2, W_cls, b_cls):
    src = edge_index[0].astype(jnp.int32)
    dst = edge_index[1].astype(jnp.int32)
    xls1, xrs1 = _transform1(x, W_l1, b_l1.reshape(4, 128),
                             W_r1, b_r1.reshape(4, 128))
    ex1 = _alpha1(src, dst, att1, xls1, xrs1)
    o1, den1 = _passb1(src, dst, ex1, xls1)
    xl2, xr2 = _transform2(o1, den1, bias1.reshape(4, 128), W_l2,
                           b_l2.reshape(1, 128), W_r2, b_r2.reshape(1, 128))
    ex2 = _alpha2(src, dst, att2, xl2, xr2)
    o2, den2 = _passb2(src, dst, ex2, xl2)
    return _final(o2, den2, bias2.reshape(1, 128), W_cls, b_cls.reshape(1, 10))


# speedup vs baseline: 4.5633x; 1.4228x over previous
import functools

import jax
import jax.numpy as jnp
from jax import lax
from jax.experimental import pallas as pl
from jax.experimental.pallas import tpu as pltpu
from jax.experimental.pallas import tpu_sc as plsc

_N = 10000
_E = 320000
_NC = 2
_NS = 16
_L = 16
_B = 80
_CW = 128
_PAD = 16
_CWP = _CW + _PAD
_EPS = 1e-16
_NEG = 0.2


def _mesh():
    return plsc.VectorSubcoreMesh(core_axis_name="c", subcore_axis_name="s")


def _lanesum(v, rbuf):
    for sh in (8, 4, 2, 1):
        rbuf[pl.ds(0, _L)] = v
        rbuf[pl.ds(_L, _L)] = v
        v = v + rbuf[pl.ds(sh, _L)]
    return v


def _make_alpha(nchunks):
    ew = _E // (_NC * _NS)
    bb = 400
    sb = 80
    nsb = bb // sb
    nblk = ew // bb
    ngrp = bb // _L

    @functools.partial(
        pl.kernel,
        mesh=_mesh(),
        out_type=jax.ShapeDtypeStruct((_E,), jnp.float32),
        scratch_types=[
            pltpu.VMEM((bb,), jnp.int32),
            pltpu.VMEM((bb,), jnp.int32),
            pltpu.VMEM((bb, _CW), jnp.float32),
            pltpu.VMEM((bb, _CW), jnp.float32),
            pltpu.VMEM((bb,), jnp.float32),
            pltpu.VMEM((nchunks * _CW,), jnp.float32),
            pltpu.VMEM((2 * _L,), jnp.float32),
            pltpu.SemaphoreType.DMA,
        ],
        compiler_params=pltpu.CompilerParams(needs_layout_passes=False),
    )
    def alpha_kernel(src_h, dst_h, att_h, xls_h, xrs_h, ex_h,
                     sidx, didx, xlb, xrb, alphab, attv, rbuf, sem):
        wid = lax.axis_index("s") * _NC + lax.axis_index("c")
        pltpu.sync_copy(att_h, attv)
        iot = lax.iota(jnp.int32, _L)
        zv = jnp.zeros((_L,), jnp.float32)
        attc = [[attv[pl.ds(c * _CW + k * _L, _L)] for k in range(_CW // _L)]
                for c in range(nchunks)]

        def blk_body(blk, carry):
            base = wid * ew + blk * bb
            d1 = pltpu.async_copy(src_h.at[pl.ds(base, bb)], sidx, sem)
            d2 = pltpu.async_copy(dst_h.at[pl.ds(base, bb)], didx, sem)
            d1.wait()
            d2.wait()

            def zal(g, c2):
                alphab[pl.ds(g * _L, _L)] = zv
                return c2

            lax.fori_loop(0, ngrp, zal, 0)
            for c in range(nchunks):
                if c > 0:
                    def bump(g, c2):
                        sl = pl.ds(g * _L, _L)
                        sidx[sl] = sidx[sl] + _N
                        didx[sl] = didx[sl] + _N
                        return c2

                    lax.fori_loop(0, ngrp, bump, 0)
                descs = []
                for j in range(nsb):
                    jsl = pl.ds(j * sb, sb)
                    descs.append(pltpu.async_copy(
                        xls_h.at[sidx.at[jsl]], xlb.at[jsl], sem))
                    descs.append(pltpu.async_copy(
                        xrs_h.at[didx.at[jsl]], xrb.at[jsl], sem))
                for d in descs:
                    d.wait()

                def grp_body(g2, carry2, c=c):
                    av = alphab[pl.ds(g2 * _L, _L)]
                    for lane in range(_L):
                        e2 = g2 * _L + lane
                        accv = jnp.zeros((_L,), jnp.float32)
                        for k in range(_CW // _L):
                            sl = pl.ds(k * _L, _L)
                            z = xlb[e2, sl] + xrb[e2, sl]
                            z = jnp.maximum(z, _NEG * z)
                            accv = accv + attc[c][k] * z
                        accv = _lanesum(accv, rbuf)
                        av = av + jnp.where(iot == lane, accv, 0.0)
                    alphab[pl.ds(g2 * _L, _L)] = av
                    return carry2

                lax.fori_loop(0, ngrp, grp_body, 0)

            def expb(g, c2):
                sl = pl.ds(g * _L, _L)
                alphab[sl] = jnp.exp(alphab[sl])
                return c2

            lax.fori_loop(0, ngrp, expb, 0)
            pltpu.sync_copy(alphab, ex_h.at[pl.ds(base, bb)])
            return carry

        lax.fori_loop(0, nblk, blk_body, 0)

    return alpha_kernel


def _make_passb(npass, col_split):
    if col_split:
        ew = _E // _NS
    else:
        ew = _E // (_NC * _NS)
    bb = 400
    sb = 80
    nsb = bb // sb
    nblk = ew // bb
    ngrp = bb // _L
    rpt = 640
    rstride = 624
    zr = 40
    nchunks_out = npass * _NC if col_split else _NC
    nden = _N if col_split else _NC * _N

    @functools.partial(
        pl.kernel,
        mesh=_mesh(),
        out_type=[
            jax.ShapeDtypeStruct((nchunks_out * _N, _CW), jnp.float32),
            jax.ShapeDtypeStruct((nden, _CW), jnp.float32),
        ],
        scratch_types=[
            pltpu.VMEM((bb,), jnp.int32),
            pltpu.VMEM((bb,), jnp.int32),
            pltpu.VMEM((nsb, sb), jnp.int32),
            pltpu.VMEM((bb,), jnp.float32),
            pltpu.VMEM((3 * sb, _CW), jnp.float32),
            pltpu.VMEM((zr, _CW), jnp.float32),
            pltpu.VMEM((rpt,), jnp.float32),
            pltpu.VMEM((_L, _CW), jnp.float32),
            pltpu.VMEM_SHARED((_N, _CW), jnp.float32),
            pltpu.VMEM_SHARED((_N,), jnp.float32),
            pltpu.SemaphoreType.DMA,
        ],
        compiler_params=pltpu.CompilerParams(needs_layout_passes=False),
    )
    def passb_kernel(src_h, dst_h, ex_h, xls_h, out_h, den_h,
                     sidx, didx, didx2, exb, xlb, zbuf, dden, dbb,
                     accsp, denslab, sem):
        core = lax.axis_index("c")
        s = lax.axis_index("s")
        rowbase = s * rstride
        zv = jnp.zeros((_L,), jnp.float32)

        def zrow(r, carry):
            for k in range(_CW // _L):
                zbuf[r, pl.ds(k * _L, _L)] = zv
            return carry

        lax.fori_loop(0, zr, zrow, 0)

        def zden(i, carry):
            dden[pl.ds(i * _L, _L)] = zv
            return carry

        lax.fori_loop(0, rpt // _L, zden, 0)
        pltpu.sync_copy(dden, denslab.at[pl.ds(rowbase, rpt)])

        for p in range(npass):
            if col_split:
                chunk = core * npass + p
            else:
                chunk = core * 0
            rowoff = chunk * _N

            def zb(i, carry):
                pltpu.sync_copy(zbuf, accsp.at[pl.ds(rowbase + i * zr, zr)])
                return carry

            lax.fori_loop(0, rpt // zr, zb, 0)
            plsc.subcore_barrier()

            def blk_body(blk, carry, rowoff=rowoff, p=p):
                if col_split:
                    base = s * ew + blk * bb
                else:
                    base = core * (_E // _NC) + s * ew + blk * bb
                d1 = pltpu.async_copy(src_h.at[pl.ds(base, bb)], sidx, sem)
                d2 = pltpu.async_copy(dst_h.at[pl.ds(base, bb)], didx, sem)
                d3 = pltpu.async_copy(ex_h.at[pl.ds(base, bb)], exb, sem)
                d1.wait()
                d2.wait()
                d3.wait()

                def prep(g, c2, rowoff=rowoff):
                    sl = pl.ds(g * _L, _L)
                    dv = didx[sl]
                    j = g // (sb // _L)
                    didx2[j, pl.ds((g % (sb // _L)) * _L, _L)] = dv
                    if col_split:
                        sidx[sl] = sidx[sl] + rowoff
                    return c2

                for g in range(ngrp):
                    prep(g, 0)
                for j0, jn in ((0, 2), (2, 3)):
                    estart = j0 * sb
                    descs = []
                    for j in range(j0, j0 + jn):
                        jsl = pl.ds(j * sb, sb)
                        bsl = pl.ds((j - j0) * sb, sb)
                        descs.append(pltpu.async_copy(
                            xls_h.at[sidx.at[jsl]], xlb.at[bsl], sem))
                    for d in descs:
                        d.wait()

                    def ebody(g2, carry2, estart=estart):
                        exv = exb[pl.ds(estart + g2 * _L, _L)]
                        for lane in range(_L):
                            e2 = g2 * _L + lane
                            exs = exv[lane]
                            for k in range(_CW // _L):
                                sl = pl.ds(k * _L, _L)
                                xlb[e2, sl] = xlb[e2, sl] * exs
                        return carry2

                    lax.fori_loop(0, jn * sb // _L, ebody, 0)
                    for j in range(j0, j0 + jn):
                        jsl = pl.ds(j * sb, sb)
                        bsl = pl.ds((j - j0) * sb, sb)
                        pltpu.sync_copy(xlb.at[bsl], accsp.at[didx2.at[j]],
                                        add=True)
                        if p == 0:
                            pltpu.sync_copy(exb.at[jsl],
                                            denslab.at[didx2.at[j]], add=True)
                return carry

            lax.fori_loop(0, nblk, blk_body, 0)
            plsc.subcore_barrier()

            outrow = rowoff + rowbase if col_split else core * _N + rowbase
            pltpu.sync_copy(accsp.at[pl.ds(rowbase, rpt)],
                            out_h.at[pl.ds(outrow, rpt)])

            if p == 0:
                def den_stage():
                    pltpu.sync_copy(denslab.at[pl.ds(rowbase, rpt)], dden)
                    denrow = rowbase if col_split else core * _N + rowbase

                    def dbc(g, carry):
                        dv = dden[pl.ds(g * _L, _L)]
                        for lane in range(_L):
                            bc = zv + dv[lane]
                            for k in range(_CW // _L):
                                dbb[lane, pl.ds(k * _L, _L)] = bc
                        pltpu.sync_copy(
                            dbb, den_h.at[pl.ds(denrow + g * _L, _L)])
                        return carry

                    lax.fori_loop(0, rpt // _L, dbc, 0)

                if col_split:
                    @pl.when(core == 0)
                    def _():
                        den_stage()
                else:
                    den_stage()
            plsc.subcore_barrier()

    return passb_kernel


def _transform1(x, W_l, b_l, W_r, b_r):
    nb = 1000
    grid = (4, _N // nb)

    def body(x_ref, wl_ref, bl_ref, wr_ref, br_ref, xl_ref, xr_ref):
        c = pl.program_id(0)
        xb = x_ref[...]
        bl = bl_ref[pl.ds(c, 1), :]
        br = br_ref[pl.ds(c, 1), :]
        xl_ref[...] = jnp.dot(xb, wl_ref[...],
                              preferred_element_type=jnp.float32) + bl
        xr_ref[...] = jnp.dot(xb, wr_ref[...],
                              preferred_element_type=jnp.float32) + br

    out = pl.pallas_call(
        body,
        grid=grid,
        in_specs=[
            pl.BlockSpec((nb, 128), lambda c, i: (i, 0)),
            pl.BlockSpec((128, 128), lambda c, i: (0, c)),
            pl.BlockSpec((4, 128), lambda c, i: (0, 0)),
            pl.BlockSpec((128, 128), lambda c, i: (0, c)),
            pl.BlockSpec((4, 128), lambda c, i: (0, 0)),
        ],
        out_specs=[
            pl.BlockSpec((nb, 128), lambda c, i: (c * grid[1] + i, 0)),
            pl.BlockSpec((nb, 128), lambda c, i: (c * grid[1] + i, 0)),
        ],
        out_shape=[
            jax.ShapeDtypeStruct((4 * _N, 128), jnp.float32),
            jax.ShapeDtypeStruct((4 * _N, 128), jnp.float32),
        ],
    )(x, W_l, b_l, W_r, b_r)
    return out


def _transform2(o1, den1, bias1, W_l2, b_l2, W_r2, b_r2):
    nb = 1000
    grid = (_N // nb,)

    def body(o0, o1c, o2c, o3c, d_ref, b1_ref, wl_ref, bl_ref, wr_ref, br_ref,
             xl_ref, xr_ref):
        den = d_ref[...] + _EPS
        acc_l = jnp.zeros((nb, 128), jnp.float32)
        acc_r = jnp.zeros((nb, 128), jnp.float32)
        for c, oc in enumerate((o0, o1c, o2c, o3c)):
            h = jnp.maximum(oc[...] / den + b1_ref[c:c + 1, :], 0.0)
            acc_l += jnp.dot(h, wl_ref[c * 128:(c + 1) * 128, :],
                             preferred_element_type=jnp.float32)
            acc_r += jnp.dot(h, wr_ref[c * 128:(c + 1) * 128, :],
                             preferred_element_type=jnp.float32)
        xl_ref[...] = acc_l + bl_ref[...]
        xr_ref[...] = acc_r + br_ref[...]

    g1 = grid[0]
    in_specs = [pl.BlockSpec((nb, 128), lambda i, c=c: (c * g1 + i, 0))
                for c in range(4)]
    in_specs += [
        pl.BlockSpec((nb, 128), lambda i: (i, 0)),
        pl.BlockSpec((4, 128), lambda i: (0, 0)),
        pl.BlockSpec((512, 128), lambda i: (0, 0)),
        pl.BlockSpec((1, 128), lambda i: (0, 0)),
        pl.BlockSpec((512, 128), lambda i: (0, 0)),
        pl.BlockSpec((1, 128), lambda i: (0, 0)),
    ]
    return pl.pallas_call(
        body,
        grid=grid,
        in_specs=in_specs,
        out_specs=[
            pl.BlockSpec((nb, 128), lambda i: (i, 0)),
            pl.BlockSpec((nb, 128), lambda i: (i, 0)),
        ],
        out_shape=[
            jax.ShapeDtypeStruct((_N, 128), jnp.float32),
            jax.ShapeDtypeStruct((_N, 128), jnp.float32),
        ],
    )(o1, o1, o1, o1, den1, bias1, W_l2, b_l2, W_r2, b_r2)


def _final(o2, den2, bias2, W_cls, b_cls):
    nb = 1000
    grid = (_N // nb,)
    g1 = grid[0]

    def body(p0, p1, d0, d1, b2_ref, wc_ref, bc_ref, out_ref, acc):
        i = pl.program_id(0)
        num = p0[...] + p1[...]
        den = d0[...] + d1[...] + _EPS
        h = jnp.maximum(num / den + b2_ref[...], 0.0)
        psum = jnp.sum(h, axis=0, keepdims=True)

        @pl.when(i == 0)
        def _():
            acc[...] = psum

        @pl.when(i > 0)
        def _():
            acc[...] = acc[...] + psum

        @pl.when(i == g1 - 1)
        def _():
            pooled = acc[...] / float(_N)
            logits = jnp.dot(pooled, wc_ref[...],
                             preferred_element_type=jnp.float32) + bc_ref[...]
            out_ref[...] = jax.nn.sigmoid(logits)

    return pl.pallas_call(
        body,
        grid=grid,
        in_specs=[
            pl.BlockSpec((nb, 128), lambda i: (i, 0)),
            pl.BlockSpec((nb, 128), lambda i: (g1 + i, 0)),
            pl.BlockSpec((nb, 128), lambda i: (i, 0)),
            pl.BlockSpec((nb, 128), lambda i: (g1 + i, 0)),
            pl.BlockSpec((1, 128), lambda i: (0, 0)),
            pl.BlockSpec((128, 10), lambda i: (0, 0)),
            pl.BlockSpec((1, 10), lambda i: (0, 0)),
        ],
        out_specs=pl.BlockSpec((1, 10), lambda i: (0, 0)),
        out_shape=jax.ShapeDtypeStruct((1, 10), jnp.float32),
        scratch_shapes=[pltpu.VMEM((1, 128), jnp.float32)],
    )(o2, o2, den2, den2, bias2, W_cls, b_cls)


_alpha1 = _make_alpha(4)
_alpha2 = _make_alpha(1)
_passb1 = _make_passb(2, True)
_passb2 = _make_passb(1, False)


def kernel(x, edge_index, W_l1, b_l1, W_r1, b_r1, att1, bias1,
           W_l2, b_l2, W_r2, b_r2, att2, bias2, W_cls, b_cls):
    src = edge_index[0].astype(jnp.int32)
    dst = edge_index[1].astype(jnp.int32)
    xls1, xrs1 = _transform1(x, W_l1, b_l1.reshape(4, 128),
                             W_r1, b_r1.reshape(4, 128))
    ex1 = _alpha1(src, dst, att1, xls1, xrs1)
    o1, den1 = _passb1(src, dst, ex1, xls1)
    xl2, xr2 = _transform2(o1, den1, bias1.reshape(4, 128), W_l2,
                           b_l2.reshape(1, 128), W_r2, b_r2.reshape(1, 128))
    ex2 = _alpha2(src, dst, att2, xl2, xr2)
    o2, den2 = _passb2(src, dst, ex2, xl2)
    return _final(o2, den2, bias2.reshape(1, 128), W_cls, b_cls.reshape(1, 10))


# speedup vs baseline: 5.2963x; 1.1606x over previous
import functools

import jax
import jax.numpy as jnp
from jax import lax
from jax.experimental import pallas as pl
from jax.experimental.pallas import tpu as pltpu
from jax.experimental.pallas import tpu_sc as plsc

_N = 10000
_E = 320000
_NC = 2
_NS = 16
_L = 16
_B = 80
_CW = 128
_PAD = 16
_CWP = _CW + _PAD
_EPS = 1e-16
_NEG = 0.2


def _mesh():
    return plsc.VectorSubcoreMesh(core_axis_name="c", subcore_axis_name="s")


def _lanesum(v, rbuf):
    for sh in (8, 4, 2, 1):
        rbuf[pl.ds(0, _L)] = v
        rbuf[pl.ds(_L, _L)] = v
        v = v + rbuf[pl.ds(sh, _L)]
    return v


def _make_alpha(nchunks):
    ew = _E // (_NC * _NS)
    b = _B
    nblk = ew // b
    ngrp = b // _L

    @functools.partial(
        pl.kernel,
        mesh=_mesh(),
        out_type=jax.ShapeDtypeStruct((_E,), jnp.float32),
        scratch_types=[
            pltpu.VMEM((4 * b,), jnp.int32),
            pltpu.VMEM((nchunks, b), jnp.int32),
            pltpu.VMEM((nchunks, b), jnp.int32),
            pltpu.VMEM((2 * b, _CW), jnp.float32),
            pltpu.VMEM((2 * b, _CW), jnp.float32),
            pltpu.VMEM((2 * b,), jnp.float32),
            pltpu.VMEM((nchunks * _CW,), jnp.float32),
            pltpu.VMEM((2 * _L,), jnp.float32),
            pltpu.SemaphoreType.DMA,
            pltpu.SemaphoreType.DMA,
            pltpu.SemaphoreType.DMA,
            pltpu.SemaphoreType.DMA,
        ],
        compiler_params=pltpu.CompilerParams(needs_layout_passes=False),
    )
    def alpha_kernel(src_h, dst_h, att_h, xls_h, xrs_h, ex_h,
                     raw, idxs, idxd, xlb, xrb, alphab, attv, rbuf,
                     sem_a, sem_b, sem_i, sem_e):
        wid = lax.axis_index("s") * _NC + lax.axis_index("c")
        pltpu.sync_copy(att_h, attv)
        iot = lax.iota(jnp.int32, _L)
        attc = [[attv[pl.ds(c * _CW + k * _L, _L)] for k in range(_CW // _L)]
                for c in range(nchunks)]
        sems = (sem_a, sem_b)

        def fire_idx(blk1, slot):
            bn = jnp.minimum(wid * ew + blk1 * b, _E - b)
            pltpu.async_copy(src_h.at[pl.ds(bn, b)],
                             raw.at[pl.ds(slot * 2 * b, b)], sem_i)
            pltpu.async_copy(dst_h.at[pl.ds(bn, b)],
                             raw.at[pl.ds(slot * 2 * b + b, b)], sem_i)

        def wait_idx():
            for _ in range(2):
                pltpu.make_async_copy(src_h.at[pl.ds(0, b)],
                                      raw.at[pl.ds(0, b)], sem_i).wait()

        def fill_idx(slot):
            for g in range(ngrp):
                sl = pl.ds(g * _L, _L)
                sv = raw[pl.ds(slot * 2 * b + g * _L, _L)]
                dv = raw[pl.ds(slot * 2 * b + b + g * _L, _L)]
                for c in range(nchunks):
                    idxs[c, sl] = sv + c * _N
                    idxd[c, sl] = dv + c * _N

        def fire_gather(c, par):
            po = pl.ds(par * b, b)
            pltpu.async_copy(xls_h.at[idxs.at[c]], xlb.at[po], sems[par])
            pltpu.async_copy(xrs_h.at[idxd.at[c]], xrb.at[po], sems[par])

        def wait_gather(par):
            po = pl.ds(0, b)
            pltpu.make_async_copy(xls_h.at[idxs.at[0]], xlb.at[po],
                                  sems[par]).wait()
            pltpu.make_async_copy(xls_h.at[idxs.at[0]], xrb.at[po],
                                  sems[par]).wait()

        def wait_ex():
            pltpu.make_async_copy(alphab.at[pl.ds(0, b)],
                                  ex_h.at[pl.ds(0, b)], sem_e).wait()

        def compute(c, par, aslot, first):
            def grp_body(g2, carry2):
                asl = pl.ds(aslot * b + g2 * _L, _L)
                av = jnp.zeros((_L,), jnp.float32) if first else alphab[asl]
                for lane in range(_L):
                    e2 = par * b + g2 * _L + lane
                    accv = jnp.zeros((_L,), jnp.float32)
                    for k in range(_CW // _L):
                        sl = pl.ds(k * _L, _L)
                        z = xlb[e2, sl] + xrb[e2, sl]
                        z = jnp.maximum(z, _NEG * z)
                        accv = accv + attc[c][k] * z
                    accv = _lanesum(accv, rbuf)
                    av = av + jnp.where(iot == lane, accv, 0.0)
                alphab[asl] = av
                return carry2

            lax.fori_loop(0, ngrp, grp_body, 0)

        def finish(blk, aslot):
            def expb(g, c2):
                sl = pl.ds(aslot * b + g * _L, _L)
                alphab[sl] = jnp.exp(alphab[sl])
                return c2

            lax.fori_loop(0, ngrp, expb, 0)
            pltpu.async_copy(alphab.at[pl.ds(aslot * b, b)],
                             ex_h.at[pl.ds(wid * ew + blk * b, b)], sem_e)

        pltpu.sync_copy(src_h.at[pl.ds(wid * ew, b)], raw.at[pl.ds(0, b)])
        pltpu.sync_copy(dst_h.at[pl.ds(wid * ew, b)], raw.at[pl.ds(b, b)])
        fill_idx(0)
        fire_gather(0, 0)
        pltpu.async_copy(ex_h.at[pl.ds(0, b)], alphab.at[pl.ds(0, b)], sem_e)

        if nchunks > 1:
            def body(blk, carry):
                slot1 = lax.rem(blk + 1, 2)
                fire_idx(blk + 1, slot1)
                for c in range(nchunks):
                    par = c % 2
                    wait_gather(par)
                    if c == 0:
                        wait_ex()
                    if c + 1 < nchunks:
                        fire_gather(c + 1, (c + 1) % 2)
                    else:
                        wait_idx()
                        fill_idx(slot1)
                        fire_gather(0, 0)
                    compute(c, par, 0, c == 0)
                finish(blk, 0)
                return carry

            lax.fori_loop(0, nblk, body, 0)
            wait_ex()
            wait_gather(0)
        else:
            def half(blk, par, slot1):
                fire_idx(blk + 1, slot1)
                wait_gather(par)
                wait_ex()
                wait_idx()
                fill_idx(slot1)
                fire_gather(0, 1 - par)
                compute(0, par, par, True)
                finish(blk, par)

            def body(i, carry):
                half(2 * i, 0, 1)
                half(2 * i + 1, 1, 0)
                return carry

            lax.fori_loop(0, nblk // 2, body, 0)
            if nblk % 2:
                half(nblk - 1, 0, 1)
                tailpar = 1
            else:
                tailpar = 0
            wait_ex()
            wait_gather(tailpar)

    return alpha_kernel


def _make_passb(npass, col_split):
    if col_split:
        ew = _E // _NS
    else:
        ew = _E // (_NC * _NS)
    bb = 400
    sb = 80
    nsb = bb // sb
    nblk = ew // bb
    ngrp = bb // _L
    rpt = 640
    rstride = 624
    zr = 40
    nchunks_out = npass * _NC if col_split else _NC
    nden = _N if col_split else _NC * _N

    @functools.partial(
        pl.kernel,
        mesh=_mesh(),
        out_type=[
            jax.ShapeDtypeStruct((nchunks_out * _N, _CW), jnp.float32),
            jax.ShapeDtypeStruct((nden, _CW), jnp.float32),
        ],
        scratch_types=[
            pltpu.VMEM((bb,), jnp.int32),
            pltpu.VMEM((bb,), jnp.int32),
            pltpu.VMEM((nsb, sb), jnp.int32),
            pltpu.VMEM((bb,), jnp.float32),
            pltpu.VMEM((3 * sb, _CW), jnp.float32),
            pltpu.VMEM((zr, _CW), jnp.float32),
            pltpu.VMEM((rpt,), jnp.float32),
            pltpu.VMEM((_L, _CW), jnp.float32),
            pltpu.VMEM_SHARED((_N, _CW), jnp.float32),
            pltpu.VMEM_SHARED((_N,), jnp.float32),
            pltpu.SemaphoreType.DMA,
        ],
        compiler_params=pltpu.CompilerParams(needs_layout_passes=False),
    )
    def passb_kernel(src_h, dst_h, ex_h, xls_h, out_h, den_h,
                     sidx, didx, didx2, exb, xlb, zbuf, dden, dbb,
                     accsp, denslab, sem):
        core = lax.axis_index("c")
        s = lax.axis_index("s")
        rowbase = s * rstride
        zv = jnp.zeros((_L,), jnp.float32)

        def zrow(r, carry):
            for k in range(_CW // _L):
                zbuf[r, pl.ds(k * _L, _L)] = zv
            return carry

        lax.fori_loop(0, zr, zrow, 0)

        def zden(i, carry):
            dden[pl.ds(i * _L, _L)] = zv
            return carry

        lax.fori_loop(0, rpt // _L, zden, 0)
        pltpu.sync_copy(dden, denslab.at[pl.ds(rowbase, rpt)])

        for p in range(npass):
            if col_split:
                chunk = core * npass + p
            else:
                chunk = core * 0
            rowoff = chunk * _N

            def zb(i, carry):
                pltpu.sync_copy(zbuf, accsp.at[pl.ds(rowbase + i * zr, zr)])
                return carry

            lax.fori_loop(0, rpt // zr, zb, 0)
            plsc.subcore_barrier()

            def blk_body(blk, carry, rowoff=rowoff, p=p):
                if col_split:
                    base = s * ew + blk * bb
                else:
                    base = core * (_E // _NC) + s * ew + blk * bb
                d1 = pltpu.async_copy(src_h.at[pl.ds(base, bb)], sidx, sem)
                d2 = pltpu.async_copy(dst_h.at[pl.ds(base, bb)], didx, sem)
                d3 = pltpu.async_copy(ex_h.at[pl.ds(base, bb)], exb, sem)
                d1.wait()
                d2.wait()
                d3.wait()

                def prep(g, c2, rowoff=rowoff):
                    sl = pl.ds(g * _L, _L)
                    dv = didx[sl]
                    j = g // (sb // _L)
                    didx2[j, pl.ds((g % (sb // _L)) * _L, _L)] = dv
                    if col_split:
                        sidx[sl] = sidx[sl] + rowoff
                    return c2

                for g in range(ngrp):
                    prep(g, 0)
                for j0, jn in ((0, 2), (2, 3)):
                    estart = j0 * sb
                    descs = []
                    for j in range(j0, j0 + jn):
                        jsl = pl.ds(j * sb, sb)
                        bsl = pl.ds((j - j0) * sb, sb)
                        descs.append(pltpu.async_copy(
                            xls_h.at[sidx.at[jsl]], xlb.at[bsl], sem))
                    for d in descs:
                        d.wait()

                    def ebody(g2, carry2, estart=estart):
                        exv = exb[pl.ds(estart + g2 * _L, _L)]
                        for lane in range(_L):
                            e2 = g2 * _L + lane
                            exs = exv[lane]
                            for k in range(_CW // _L):
                                sl = pl.ds(k * _L, _L)
                                xlb[e2, sl] = xlb[e2, sl] * exs
                        return carry2

                    lax.fori_loop(0, jn * sb // _L, ebody, 0)
                    for j in range(j0, j0 + jn):
                        jsl = pl.ds(j * sb, sb)
                        bsl = pl.ds((j - j0) * sb, sb)
                        pltpu.sync_copy(xlb.at[bsl], accsp.at[didx2.at[j]],
                                        add=True)
                        if p == 0:
                            pltpu.sync_copy(exb.at[jsl],
                                            denslab.at[didx2.at[j]], add=True)
                return carry

            lax.fori_loop(0, nblk, blk_body, 0)
            plsc.subcore_barrier()

            outrow = rowoff + rowbase if col_split else core * _N + rowbase
            pltpu.sync_copy(accsp.at[pl.ds(rowbase, rpt)],
                            out_h.at[pl.ds(outrow, rpt)])

            if p == 0:
                def den_stage():
                    pltpu.sync_copy(denslab.at[pl.ds(rowbase, rpt)], dden)
                    denrow = rowbase if col_split else core * _N + rowbase

                    def dbc(g, carry):
                        dv = dden[pl.ds(g * _L, _L)]
                        for lane in range(_L):
                            bc = zv + dv[lane]
                            for k in range(_CW // _L):
                                dbb[lane, pl.ds(k * _L, _L)] = bc
                        pltpu.sync_copy(
                            dbb, den_h.at[pl.ds(denrow + g * _L, _L)])
                        return carry

                    lax.fori_loop(0, rpt // _L, dbc, 0)

                if col_split:
                    @pl.when(core == 0)
                    def _():
                        den_stage()
                else:
                    den_stage()
            plsc.subcore_barrier()

    return passb_kernel


def _transform1(x, W_l, b_l, W_r, b_r):
    nb = 1000
    grid = (4, _N // nb)

    def body(x_ref, wl_ref, bl_ref, wr_ref, br_ref, xl_ref, xr_ref):
        c = pl.program_id(0)
        xb = x_ref[...]
        bl = bl_ref[pl.ds(c, 1), :]
        br = br_ref[pl.ds(c, 1), :]
        xl_ref[...] = jnp.dot(xb, wl_ref[...],
                              preferred_element_type=jnp.float32) + bl
        xr_ref[...] = jnp.dot(xb, wr_ref[...],
                              preferred_element_type=jnp.float32) + br

    out = pl.pallas_call(
        body,
        grid=grid,
        in_specs=[
            pl.BlockSpec((nb, 128), lambda c, i: (i, 0)),
            pl.BlockSpec((128, 128), lambda c, i: (0, c)),
            pl.BlockSpec((4, 128), lambda c, i: (0, 0)),
            pl.BlockSpec((128, 128), lambda c, i: (0, c)),
            pl.BlockSpec((4, 128), lambda c, i: (0, 0)),
        ],
        out_specs=[
            pl.BlockSpec((nb, 128), lambda c, i: (c * grid[1] + i, 0)),
            pl.BlockSpec((nb, 128), lambda c, i: (c * grid[1] + i, 0)),
        ],
        out_shape=[
            jax.ShapeDtypeStruct((4 * _N, 128), jnp.float32),
            jax.ShapeDtypeStruct((4 * _N, 128), jnp.float32),
        ],
    )(x, W_l, b_l, W_r, b_r)
    return out


def _transform2(o1, den1, bias1, W_l2, b_l2, W_r2, b_r2):
    nb = 1000
    grid = (_N // nb,)

    def body(o0, o1c, o2c, o3c, d_ref, b1_ref, wl_ref, bl_ref, wr_ref, br_ref,
             xl_ref, xr_ref):
        den = d_ref[...] + _EPS
        acc_l = jnp.zeros((nb, 128), jnp.float32)
        acc_r = jnp.zeros((nb, 128), jnp.float32)
        for c, oc in enumerate((o0, o1c, o2c, o3c)):
            h = jnp.maximum(oc[...] / den + b1_ref[c:c + 1, :], 0.0)
            acc_l += jnp.dot(h, wl_ref[c * 128:(c + 1) * 128, :],
                             preferred_element_type=jnp.float32)
            acc_r += jnp.dot(h, wr_ref[c * 128:(c + 1) * 128, :],
                             preferred_element_type=jnp.float32)
        xl_ref[...] = acc_l + bl_ref[...]
        xr_ref[...] = acc_r + br_ref[...]

    g1 = grid[0]
    in_specs = [pl.BlockSpec((nb, 128), lambda i, c=c: (c * g1 + i, 0))
                for c in range(4)]
    in_specs += [
        pl.BlockSpec((nb, 128), lambda i: (i, 0)),
        pl.BlockSpec((4, 128), lambda i: (0, 0)),
        pl.BlockSpec((512, 128), lambda i: (0, 0)),
        pl.BlockSpec((1, 128), lambda i: (0, 0)),
        pl.BlockSpec((512, 128), lambda i: (0, 0)),
        pl.BlockSpec((1, 128), lambda i: (0, 0)),
    ]
    return pl.pallas_call(
        body,
        grid=grid,
        in_specs=in_specs,
        out_specs=[
            pl.BlockSpec((nb, 128), lambda i: (i, 0)),
            pl.BlockSpec((nb, 128), lambda i: (i, 0)),
        ],
        out_shape=[
            jax.ShapeDtypeStruct((_N, 128), jnp.float32),
            jax.ShapeDtypeStruct((_N, 128), jnp.float32),
        ],
    )(o1, o1, o1, o1, den1, bias1, W_l2, b_l2, W_r2, b_r2)


def _final(o2, den2, bias2, W_cls, b_cls):
    nb = 1000
    grid = (_N // nb,)
    g1 = grid[0]

    def body(p0, p1, d0, d1, b2_ref, wc_ref, bc_ref, out_ref, acc):
        i = pl.program_id(0)
        num = p0[...] + p1[...]
        den = d0[...] + d1[...] + _EPS
        h = jnp.maximum(num / den + b2_ref[...], 0.0)
        psum = jnp.sum(h, axis=0, keepdims=True)

        @pl.when(i == 0)
        def _():
            acc[...] = psum

        @pl.when(i > 0)
        def _():
            acc[...] = acc[...] + psum

        @pl.when(i == g1 - 1)
        def _():
            pooled = acc[...] / float(_N)
            logits = jnp.dot(pooled, wc_ref[...],
                             preferred_element_type=jnp.float32) + bc_ref[...]
            out_ref[...] = jax.nn.sigmoid(logits)

    return pl.pallas_call(
        body,
        grid=grid,
        in_specs=[
            pl.BlockSpec((nb, 128), lambda i: (i, 0)),
            pl.BlockSpec((nb, 128), lambda i: (g1 + i, 0)),
            pl.BlockSpec((nb, 128), lambda i: (i, 0)),
            pl.BlockSpec((nb, 128), lambda i: (g1 + i, 0)),
            pl.BlockSpec((1, 128), lambda i: (0, 0)),
            pl.BlockSpec((128, 10), lambda i: (0, 0)),
            pl.BlockSpec((1, 10), lambda i: (0, 0)),
        ],
        out_specs=pl.BlockSpec((1, 10), lambda i: (0, 0)),
        out_shape=jax.ShapeDtypeStruct((1, 10), jnp.float32),
        scratch_shapes=[pltpu.VMEM((1, 128), jnp.float32)],
    )(o2, o2, den2, den2, bias2, W_cls, b_cls)


_alpha1 = _make_alpha(4)
_alpha2 = _make_alpha(1)
_passb1 = _make_passb(2, True)
_passb2 = _make_passb(1, False)


def kernel(x, edge_index, W_l1, b_l1, W_r1, b_r1, att1, bias1,
           W_l2, b_l2, W_r2, b_r2, att2, bias2, W_cls, b_cls):
    src = edge_index[0].astype(jnp.int32)
    dst = edge_index[1].astype(jnp.int32)
    xls1, xrs1 = _transform1(x, W_l1, b_l1.reshape(4, 128),
                             W_r1, b_r1.reshape(4, 128))
    ex1 = _alpha1(src, dst, att1, xls1, xrs1)
    o1, den1 = _passb1(src, dst, ex1, xls1)
    xl2, xr2 = _transform2(o1, den1, bias1.reshape(4, 128), W_l2,
                           b_l2.reshape(1, 128), W_r2, b_r2.reshape(1, 128))
    ex2 = _alpha2(src, dst, att2, xl2, xr2)
    o2, den2 = _passb2(src, dst, ex2, xl2)
    return _final(o2, den2, bias2.reshape(1, 128), W_cls, b_cls.reshape(1, 10))


# speedup vs baseline: 5.5368x; 1.0454x over previous
import functools

import jax
import jax.numpy as jnp
from jax import lax
from jax.experimental import pallas as pl
from jax.experimental.pallas import tpu as pltpu
from jax.experimental.pallas import tpu_sc as plsc

_N = 10000
_E = 320000
_NC = 2
_NS = 16
_L = 16
_B = 80
_CW = 128
_PAD = 16
_CWP = _CW + _PAD
_EPS = 1e-16
_NEG = 0.2


def _mesh():
    return plsc.VectorSubcoreMesh(core_axis_name="c", subcore_axis_name="s")


def _lanesum(v, rbuf):
    for sh in (8, 4, 2, 1):
        rbuf[pl.ds(0, _L)] = v
        rbuf[pl.ds(_L, _L)] = v
        v = v + rbuf[pl.ds(sh, _L)]
    return v


def _make_alpha(nchunks, packed=False):
    ew = _E // (_NC * _NS)
    b = _B
    nblk = ew // b
    ngrp = b // _L

    @functools.partial(
        pl.kernel,
        mesh=_mesh(),
        out_type=jax.ShapeDtypeStruct((_E,), jnp.float32),
        scratch_types=[
            pltpu.VMEM((4 * b,), jnp.int32),
            pltpu.VMEM((nchunks, b), jnp.int32),
            pltpu.VMEM((nchunks, b), jnp.int32),
            pltpu.VMEM((2 * b, _CW), jnp.float32),
            pltpu.VMEM((2 * b, _CW), jnp.float32),
            pltpu.VMEM((2 * b,), jnp.float32),
            pltpu.VMEM((nchunks * _CW,), jnp.float32),
            pltpu.VMEM((2 * _L,), jnp.float32),
            pltpu.SemaphoreType.DMA,
            pltpu.SemaphoreType.DMA,
            pltpu.SemaphoreType.DMA,
            pltpu.SemaphoreType.DMA,
        ],
        compiler_params=pltpu.CompilerParams(needs_layout_passes=False),
    )
    def alpha_kernel(src_h, dst_h, att_h, xls_h, xrs_h, ex_h,
                     raw, idxs, idxd, xlb, xrb, alphab, attv, rbuf,
                     sem_a, sem_b, sem_i, sem_e):
        wid = lax.axis_index("s") * _NC + lax.axis_index("c")
        pltpu.sync_copy(att_h, attv)
        iot = lax.iota(jnp.int32, _L)
        if packed:
            attc = [[plsc.bitcast(attv[pl.ds(c * _CW + k * _L, _L)],
                                  jnp.bfloat16)
                     for k in range(_CW // _L)] for c in range(nchunks)]
        else:
            attc = [[attv[pl.ds(c * _CW + k * _L, _L)]
                     for k in range(_CW // _L)] for c in range(nchunks)]
        sems = (sem_a, sem_b)

        def fire_idx(blk1, slot):
            bn = jnp.minimum(wid * ew + blk1 * b, _E - b)
            pltpu.async_copy(src_h.at[pl.ds(bn, b)],
                             raw.at[pl.ds(slot * 2 * b, b)], sem_i)
            pltpu.async_copy(dst_h.at[pl.ds(bn, b)],
                             raw.at[pl.ds(slot * 2 * b + b, b)], sem_i)

        def wait_idx():
            for _ in range(2):
                pltpu.make_async_copy(src_h.at[pl.ds(0, b)],
                                      raw.at[pl.ds(0, b)], sem_i).wait()

        def fill_idx(slot):
            for g in range(ngrp):
                sl = pl.ds(g * _L, _L)
                sv = raw[pl.ds(slot * 2 * b + g * _L, _L)]
                dv = raw[pl.ds(slot * 2 * b + b + g * _L, _L)]
                for c in range(nchunks):
                    idxs[c, sl] = sv + c * _N
                    idxd[c, sl] = dv + c * _N

        def fire_gather(c, par):
            po = pl.ds(par * b, b)
            pltpu.async_copy(xls_h.at[idxs.at[c]], xlb.at[po], sems[par])
            pltpu.async_copy(xrs_h.at[idxd.at[c]], xrb.at[po], sems[par])

        def wait_gather(par):
            po = pl.ds(0, b)
            pltpu.make_async_copy(xls_h.at[idxs.at[0]], xlb.at[po],
                                  sems[par]).wait()
            pltpu.make_async_copy(xls_h.at[idxs.at[0]], xrb.at[po],
                                  sems[par]).wait()

        def wait_ex():
            pltpu.make_async_copy(alphab.at[pl.ds(0, b)],
                                  ex_h.at[pl.ds(0, b)], sem_e).wait()

        def compute(c, par, aslot, first):
            def grp_body(g2, carry2):
                asl = pl.ds(aslot * b + g2 * _L, _L)
                av = jnp.zeros((_L,), jnp.float32) if first else alphab[asl]
                for lane in range(_L):
                    e2 = par * b + g2 * _L + lane
                    accv = jnp.zeros((_L,), jnp.float32)
                    for k in range(_CW // _L):
                        sl = pl.ds(k * _L, _L)
                        if packed:
                            zl = plsc.bitcast(xlb[e2, sl], jnp.bfloat16)
                            zr = plsc.bitcast(xrb[e2, sl], jnp.bfloat16)
                            z = zl + zr
                            z = jnp.maximum(z, z * _NEG)
                            m = z * attc[c][k]
                            m0, m1 = plsc.unpack(
                                m, format=plsc.PackFormat.INTERLEAVED)
                            accv = accv + m0 + m1
                        else:
                            z = xlb[e2, sl] + xrb[e2, sl]
                            z = jnp.maximum(z, _NEG * z)
                            accv = accv + attc[c][k] * z
                    accv = _lanesum(accv, rbuf)
                    av = av + jnp.where(iot == lane, accv, 0.0)
                alphab[asl] = av
                return carry2

            lax.fori_loop(0, ngrp, grp_body, 0)

        def finish(blk, aslot):
            def expb(g, c2):
                sl = pl.ds(aslot * b + g * _L, _L)
                alphab[sl] = jnp.exp(alphab[sl])
                return c2

            lax.fori_loop(0, ngrp, expb, 0)
            pltpu.async_copy(alphab.at[pl.ds(aslot * b, b)],
                             ex_h.at[pl.ds(wid * ew + blk * b, b)], sem_e)

        pltpu.sync_copy(src_h.at[pl.ds(wid * ew, b)], raw.at[pl.ds(0, b)])
        pltpu.sync_copy(dst_h.at[pl.ds(wid * ew, b)], raw.at[pl.ds(b, b)])
        fill_idx(0)
        fire_gather(0, 0)
        pltpu.async_copy(ex_h.at[pl.ds(0, b)], alphab.at[pl.ds(0, b)], sem_e)

        if nchunks > 1:
            def body(blk, carry):
                slot1 = lax.rem(blk + 1, 2)
                fire_idx(blk + 1, slot1)
                for c in range(nchunks):
                    par = c % 2
                    wait_gather(par)
                    if c == 0:
                        wait_ex()
                    if c + 1 < nchunks:
                        fire_gather(c + 1, (c + 1) % 2)
                    else:
                        wait_idx()
                        fill_idx(slot1)
                        fire_gather(0, 0)
                    compute(c, par, 0, c == 0)
                finish(blk, 0)
                return carry

            lax.fori_loop(0, nblk, body, 0)
            wait_ex()
            wait_gather(0)
        else:
            def half(blk, par, slot1):
                fire_idx(blk + 1, slot1)
                wait_gather(par)
                wait_ex()
                wait_idx()
                fill_idx(slot1)
                fire_gather(0, 1 - par)
                compute(0, par, par, True)
                finish(blk, par)

            def body(i, carry):
                half(2 * i, 0, 1)
                half(2 * i + 1, 1, 0)
                return carry

            lax.fori_loop(0, nblk // 2, body, 0)
            if nblk % 2:
                half(nblk - 1, 0, 1)
                tailpar = 1
            else:
                tailpar = 0
            wait_ex()
            wait_gather(tailpar)

    return alpha_kernel


def _make_passb(npass, col_split):
    if col_split:
        ew = _E // _NS
    else:
        ew = _E // (_NC * _NS)
    bb = 400
    sb = 80
    nsb = bb // sb
    nblk = ew // bb
    ngrp = bb // _L
    rpt = 640
    rstride = 624
    zr = 40
    nchunks_out = npass * _NC if col_split else _NC
    nden = _N if col_split else _NC * _N

    @functools.partial(
        pl.kernel,
        mesh=_mesh(),
        out_type=[
            jax.ShapeDtypeStruct((nchunks_out * _N, _CW), jnp.float32),
            jax.ShapeDtypeStruct((nden, _CW), jnp.float32),
        ],
        scratch_types=[
            pltpu.VMEM((bb,), jnp.int32),
            pltpu.VMEM((bb,), jnp.int32),
            pltpu.VMEM((nsb, sb), jnp.int32),
            pltpu.VMEM((bb,), jnp.float32),
            pltpu.VMEM((3 * sb, _CW), jnp.float32),
            pltpu.VMEM((zr, _CW), jnp.float32),
            pltpu.VMEM((rpt,), jnp.float32),
            pltpu.VMEM((_L, _CW), jnp.float32),
            pltpu.VMEM_SHARED((_N, _CW), jnp.float32),
            pltpu.VMEM_SHARED((_N,), jnp.float32),
            pltpu.SemaphoreType.DMA,
        ],
        compiler_params=pltpu.CompilerParams(needs_layout_passes=False),
    )
    def passb_kernel(src_h, dst_h, ex_h, xls_h, out_h, den_h,
                     sidx, didx, didx2, exb, xlb, zbuf, dden, dbb,
                     accsp, denslab, sem):
        core = lax.axis_index("c")
        s = lax.axis_index("s")
        rowbase = s * rstride
        zv = jnp.zeros((_L,), jnp.float32)

        def zrow(r, carry):
            for k in range(_CW // _L):
                zbuf[r, pl.ds(k * _L, _L)] = zv
            return carry

        lax.fori_loop(0, zr, zrow, 0)

        def zden(i, carry):
            dden[pl.ds(i * _L, _L)] = zv
            return carry

        lax.fori_loop(0, rpt // _L, zden, 0)
        pltpu.sync_copy(dden, denslab.at[pl.ds(rowbase, rpt)])

        for p in range(npass):
            if col_split:
                chunk = core * npass + p
            else:
                chunk = core * 0
            rowoff = chunk * _N

            def zb(i, carry):
                pltpu.sync_copy(zbuf, accsp.at[pl.ds(rowbase + i * zr, zr)])
                return carry

            lax.fori_loop(0, rpt // zr, zb, 0)
            plsc.subcore_barrier()

            def blk_body(blk, carry, rowoff=rowoff, p=p):
                if col_split:
                    base = s * ew + blk * bb
                else:
                    base = core * (_E // _NC) + s * ew + blk * bb
                d1 = pltpu.async_copy(src_h.at[pl.ds(base, bb)], sidx, sem)
                d2 = pltpu.async_copy(dst_h.at[pl.ds(base, bb)], didx, sem)
                d3 = pltpu.async_copy(ex_h.at[pl.ds(base, bb)], exb, sem)
                d1.wait()
                d2.wait()
                d3.wait()

                def prep(g, c2, rowoff=rowoff):
                    sl = pl.ds(g * _L, _L)
                    dv = didx[sl]
                    j = g // (sb // _L)
                    didx2[j, pl.ds((g % (sb // _L)) * _L, _L)] = dv
                    if col_split:
                        sidx[sl] = sidx[sl] + rowoff
                    return c2

                for g in range(ngrp):
                    prep(g, 0)
                for j0, jn in ((0, 2), (2, 3)):
                    estart = j0 * sb
                    descs = []
                    for j in range(j0, j0 + jn):
                        jsl = pl.ds(j * sb, sb)
                        bsl = pl.ds((j - j0) * sb, sb)
                        descs.append(pltpu.async_copy(
                            xls_h.at[sidx.at[jsl]], xlb.at[bsl], sem))
                    for d in descs:
                        d.wait()

                    def ebody(g2, carry2, estart=estart):
                        exv = exb[pl.ds(estart + g2 * _L, _L)]
                        for lane in range(_L):
                            e2 = g2 * _L + lane
                            exs = exv[lane]
                            for k in range(_CW // _L):
                                sl = pl.ds(k * _L, _L)
                                xlb[e2, sl] = xlb[e2, sl] * exs
                        return carry2

                    lax.fori_loop(0, jn * sb // _L, ebody, 0)
                    for j in range(j0, j0 + jn):
                        jsl = pl.ds(j * sb, sb)
                        bsl = pl.ds((j - j0) * sb, sb)
                        pltpu.sync_copy(xlb.at[bsl], accsp.at[didx2.at[j]],
                                        add=True)
                        if p == 0:
                            pltpu.sync_copy(exb.at[jsl],
                                            denslab.at[didx2.at[j]], add=True)
                return carry

            lax.fori_loop(0, nblk, blk_body, 0)
            plsc.subcore_barrier()

            outrow = rowoff + rowbase if col_split else core * _N + rowbase
            pltpu.sync_copy(accsp.at[pl.ds(rowbase, rpt)],
                            out_h.at[pl.ds(outrow, rpt)])

            if p == 0:
                def den_stage():
                    pltpu.sync_copy(denslab.at[pl.ds(rowbase, rpt)], dden)
                    denrow = rowbase if col_split else core * _N + rowbase

                    def dbc(g, carry):
                        dv = dden[pl.ds(g * _L, _L)]
                        for lane in range(_L):
                            bc = zv + dv[lane]
                            for k in range(_CW // _L):
                                dbb[lane, pl.ds(k * _L, _L)] = bc
                        pltpu.sync_copy(
                            dbb, den_h.at[pl.ds(denrow + g * _L, _L)])
                        return carry

                    lax.fori_loop(0, rpt // _L, dbc, 0)

                if col_split:
                    @pl.when(core == 0)
                    def _():
                        den_stage()
                else:
                    den_stage()
            plsc.subcore_barrier()

    return passb_kernel


def _transform1(x, W_l, b_l, W_r, b_r):
    nb = 1000
    grid = (4, _N // nb)

    def body(x_ref, wl_ref, bl_ref, wr_ref, br_ref, xl_ref, xr_ref):
        c = pl.program_id(0)
        xb = x_ref[...]
        bl = bl_ref[pl.ds(c, 1), :]
        br = br_ref[pl.ds(c, 1), :]
        xl_ref[...] = jnp.dot(xb, wl_ref[...],
                              preferred_element_type=jnp.float32) + bl
        xr_ref[...] = jnp.dot(xb, wr_ref[...],
                              preferred_element_type=jnp.float32) + br

    out = pl.pallas_call(
        body,
        grid=grid,
        in_specs=[
            pl.BlockSpec((nb, 128), lambda c, i: (i, 0)),
            pl.BlockSpec((128, 128), lambda c, i: (0, c)),
            pl.BlockSpec((4, 128), lambda c, i: (0, 0)),
            pl.BlockSpec((128, 128), lambda c, i: (0, c)),
            pl.BlockSpec((4, 128), lambda c, i: (0, 0)),
        ],
        out_specs=[
            pl.BlockSpec((nb, 128), lambda c, i: (c * grid[1] + i, 0)),
            pl.BlockSpec((nb, 128), lambda c, i: (c * grid[1] + i, 0)),
        ],
        out_shape=[
            jax.ShapeDtypeStruct((4 * _N, 128), jnp.float32),
            jax.ShapeDtypeStruct((4 * _N, 128), jnp.float32),
        ],
    )(x, W_l, b_l, W_r, b_r)
    return out


def _transform2(o1, den1, bias1, W_l2, b_l2, W_r2, b_r2):
    nb = 1000
    grid = (_N // nb,)

    def body(o0, o1c, o2c, o3c, d_ref, b1_ref, wl_ref, bl_ref, wr_ref, br_ref,
             xl_ref, xr_ref):
        den = d_ref[...] + _EPS
        acc_l = jnp.zeros((nb, 128), jnp.float32)
        acc_r = jnp.zeros((nb, 128), jnp.float32)
        for c, oc in enumerate((o0, o1c, o2c, o3c)):
            h = jnp.maximum(oc[...] / den + b1_ref[c:c + 1, :], 0.0)
            acc_l += jnp.dot(h, wl_ref[c * 128:(c + 1) * 128, :],
                             preferred_element_type=jnp.float32)
            acc_r += jnp.dot(h, wr_ref[c * 128:(c + 1) * 128, :],
                             preferred_element_type=jnp.float32)
        xl_ref[...] = acc_l + bl_ref[...]
        xr_ref[...] = acc_r + br_ref[...]

    g1 = grid[0]
    in_specs = [pl.BlockSpec((nb, 128), lambda i, c=c: (c * g1 + i, 0))
                for c in range(4)]
    in_specs += [
        pl.BlockSpec((nb, 128), lambda i: (i, 0)),
        pl.BlockSpec((4, 128), lambda i: (0, 0)),
        pl.BlockSpec((512, 128), lambda i: (0, 0)),
        pl.BlockSpec((1, 128), lambda i: (0, 0)),
        pl.BlockSpec((512, 128), lambda i: (0, 0)),
        pl.BlockSpec((1, 128), lambda i: (0, 0)),
    ]
    return pl.pallas_call(
        body,
        grid=grid,
        in_specs=in_specs,
        out_specs=[
            pl.BlockSpec((nb, 128), lambda i: (i, 0)),
            pl.BlockSpec((nb, 128), lambda i: (i, 0)),
        ],
        out_shape=[
            jax.ShapeDtypeStruct((_N, 128), jnp.float32),
            jax.ShapeDtypeStruct((_N, 128), jnp.float32),
        ],
    )(o1, o1, o1, o1, den1, bias1, W_l2, b_l2, W_r2, b_r2)


def _final(o2, den2, bias2, W_cls, b_cls):
    nb = 1000
    grid = (_N // nb,)
    g1 = grid[0]

    def body(p0, p1, d0, d1, b2_ref, wc_ref, bc_ref, out_ref, acc):
        i = pl.program_id(0)
        num = p0[...] + p1[...]
        den = d0[...] + d1[...] + _EPS
        h = jnp.maximum(num / den + b2_ref[...], 0.0)
        psum = jnp.sum(h, axis=0, keepdims=True)

        @pl.when(i == 0)
        def _():
            acc[...] = psum

        @pl.when(i > 0)
        def _():
            acc[...] = acc[...] + psum

        @pl.when(i == g1 - 1)
        def _():
            pooled = acc[...] / float(_N)
            logits = jnp.dot(pooled, wc_ref[...],
                             preferred_element_type=jnp.float32) + bc_ref[...]
            out_ref[...] = jax.nn.sigmoid(logits)

    return pl.pallas_call(
        body,
        grid=grid,
        in_specs=[
            pl.BlockSpec((nb, 128), lambda i: (i, 0)),
            pl.BlockSpec((nb, 128), lambda i: (g1 + i, 0)),
            pl.BlockSpec((nb, 128), lambda i: (i, 0)),
            pl.BlockSpec((nb, 128), lambda i: (g1 + i, 0)),
            pl.BlockSpec((1, 128), lambda i: (0, 0)),
            pl.BlockSpec((128, 10), lambda i: (0, 0)),
            pl.BlockSpec((1, 10), lambda i: (0, 0)),
        ],
        out_specs=pl.BlockSpec((1, 10), lambda i: (0, 0)),
        out_shape=jax.ShapeDtypeStruct((1, 10), jnp.float32),
        scratch_shapes=[pltpu.VMEM((1, 128), jnp.float32)],
    )(o2, o2, den2, den2, bias2, W_cls, b_cls)


_alpha1 = _make_alpha(2, packed=True)
_alpha2 = _make_alpha(1)
_passb1 = _make_passb(2, True)
_passb2 = _make_passb(1, False)


def _pack_pairs(y):
    a = y.reshape(2, 2, _N, 128).transpose(0, 2, 1, 3).reshape(2 * _N, 256)
    b = a.astype(jnp.bfloat16).reshape(2 * _N, 128, 2)
    return lax.bitcast_convert_type(b, jnp.float32)


def kernel(x, edge_index, W_l1, b_l1, W_r1, b_r1, att1, bias1,
           W_l2, b_l2, W_r2, b_r2, att2, bias2, W_cls, b_cls):
    src = edge_index[0].astype(jnp.int32)
    dst = edge_index[1].astype(jnp.int32)
    xls1, xrs1 = _transform1(x, W_l1, b_l1.reshape(4, 128),
                             W_r1, b_r1.reshape(4, 128))
    att1p = lax.bitcast_convert_type(
        att1.astype(jnp.bfloat16).reshape(256, 2), jnp.float32)
    ex1 = _alpha1(src, dst, att1p, _pack_pairs(xls1), _pack_pairs(xrs1))
    o1, den1 = _passb1(src, dst, ex1, xls1)
    xl2, xr2 = _transform2(o1, den1, bias1.reshape(4, 128), W_l2,
                           b_l2.reshape(1, 128), W_r2, b_r2.reshape(1, 128))
    ex2 = _alpha2(src, dst, att2, xl2, xr2)
    o2, den2 = _passb2(src, dst, ex2, xl2)
    return _final(o2, den2, bias2.reshape(1, 128), W_cls, b_cls.reshape(1, 10))


# speedup vs baseline: 6.2412x; 1.1272x over previous
import functools

import jax
import jax.numpy as jnp
from jax import lax
from jax.experimental import pallas as pl
from jax.experimental.pallas import tpu as pltpu
from jax.experimental.pallas import tpu_sc as plsc

_N = 10000
_E = 320000
_NC = 2
_NS = 16
_L = 16
_B = 80
_CW = 128
_PAD = 16
_CWP = _CW + _PAD
_EPS = 1e-16
_NEG = 0.2


def _mesh():
    return plsc.VectorSubcoreMesh(core_axis_name="c", subcore_axis_name="s")


def _lanesum(v, rbuf):
    for sh in (8, 4, 2, 1):
        rbuf[pl.ds(0, _L)] = v
        rbuf[pl.ds(_L, _L)] = v
        v = v + rbuf[pl.ds(sh, _L)]
    return v


def _make_alpha(nchunks, packed=False):
    ew = _E // (_NC * _NS)
    b = _B
    nblk = ew // b
    ngrp = b // _L

    @functools.partial(
        pl.kernel,
        mesh=_mesh(),
        out_type=jax.ShapeDtypeStruct((_E,), jnp.float32),
        scratch_types=[
            pltpu.VMEM((4 * b,), jnp.int32),
            pltpu.VMEM((nchunks, b), jnp.int32),
            pltpu.VMEM((nchunks, b), jnp.int32),
            pltpu.VMEM((2 * b, _CW), jnp.float32),
            pltpu.VMEM((2 * b, _CW), jnp.float32),
            pltpu.VMEM((2 * b,), jnp.float32),
            pltpu.VMEM((nchunks * _CW,), jnp.float32),
            pltpu.VMEM((2 * _L,), jnp.float32),
            pltpu.SemaphoreType.DMA,
            pltpu.SemaphoreType.DMA,
            pltpu.SemaphoreType.DMA,
            pltpu.SemaphoreType.DMA,
        ],
        compiler_params=pltpu.CompilerParams(needs_layout_passes=False),
    )
    def alpha_kernel(src_h, dst_h, att_h, xls_h, xrs_h, ex_h,
                     raw, idxs, idxd, xlb, xrb, alphab, attv, rbuf,
                     sem_a, sem_b, sem_i, sem_e):
        wid = lax.axis_index("s") * _NC + lax.axis_index("c")
        pltpu.sync_copy(att_h, attv)
        iot = lax.iota(jnp.int32, _L)
        if packed:
            attc = [[plsc.bitcast(attv[pl.ds(c * _CW + k * _L, _L)],
                                  jnp.bfloat16)
                     for k in range(_CW // _L)] for c in range(nchunks)]
        else:
            attc = [[attv[pl.ds(c * _CW + k * _L, _L)]
                     for k in range(_CW // _L)] for c in range(nchunks)]
        sems = (sem_a, sem_b)

        def fire_idx(blk1, slot):
            bn = jnp.minimum(wid * ew + blk1 * b, _E - b)
            pltpu.async_copy(src_h.at[pl.ds(bn, b)],
                             raw.at[pl.ds(slot * 2 * b, b)], sem_i)
            pltpu.async_copy(dst_h.at[pl.ds(bn, b)],
                             raw.at[pl.ds(slot * 2 * b + b, b)], sem_i)

        def wait_idx():
            for _ in range(2):
                pltpu.make_async_copy(src_h.at[pl.ds(0, b)],
                                      raw.at[pl.ds(0, b)], sem_i).wait()

        def fill_idx(slot):
            for g in range(ngrp):
                sl = pl.ds(g * _L, _L)
                sv = raw[pl.ds(slot * 2 * b + g * _L, _L)]
                dv = raw[pl.ds(slot * 2 * b + b + g * _L, _L)]
                for c in range(nchunks):
                    idxs[c, sl] = sv + c * _N
                    idxd[c, sl] = dv + c * _N

        def fire_gather(c, par):
            po = pl.ds(par * b, b)
            pltpu.async_copy(xls_h.at[idxs.at[c]], xlb.at[po], sems[par])
            pltpu.async_copy(xrs_h.at[idxd.at[c]], xrb.at[po], sems[par])

        def wait_gather(par):
            po = pl.ds(0, b)
            pltpu.make_async_copy(xls_h.at[idxs.at[0]], xlb.at[po],
                                  sems[par]).wait()
            pltpu.make_async_copy(xls_h.at[idxs.at[0]], xrb.at[po],
                                  sems[par]).wait()

        def wait_ex():
            pltpu.make_async_copy(alphab.at[pl.ds(0, b)],
                                  ex_h.at[pl.ds(0, b)], sem_e).wait()

        def compute(c, par, aslot, first):
            def grp_body(g2, carry2):
                asl = pl.ds(aslot * b + g2 * _L, _L)
                av = jnp.zeros((_L,), jnp.float32) if first else alphab[asl]
                for lane in range(_L):
                    e2 = par * b + g2 * _L + lane
                    accv = jnp.zeros((_L,), jnp.float32)
                    for k in range(_CW // _L):
                        sl = pl.ds(k * _L, _L)
                        if packed:
                            zl = plsc.bitcast(xlb[e2, sl], jnp.bfloat16)
                            zr = plsc.bitcast(xrb[e2, sl], jnp.bfloat16)
                            z = zl + zr
                            z = jnp.maximum(z, z * _NEG)
                            m = z * attc[c][k]
                            m0, m1 = plsc.unpack(
                                m, format=plsc.PackFormat.INTERLEAVED)
                            accv = accv + m0 + m1
                        else:
                            z = xlb[e2, sl] + xrb[e2, sl]
                            z = jnp.maximum(z, _NEG * z)
                            accv = accv + attc[c][k] * z
                    accv = _lanesum(accv, rbuf)
                    av = av + jnp.where(iot == lane, accv, 0.0)
                alphab[asl] = av
                return carry2

            lax.fori_loop(0, ngrp, grp_body, 0)

        def finish(blk, aslot):
            def expb(g, c2):
                sl = pl.ds(aslot * b + g * _L, _L)
                alphab[sl] = jnp.exp(alphab[sl])
                return c2

            lax.fori_loop(0, ngrp, expb, 0)
            pltpu.async_copy(alphab.at[pl.ds(aslot * b, b)],
                             ex_h.at[pl.ds(wid * ew + blk * b, b)], sem_e)

        pltpu.sync_copy(src_h.at[pl.ds(wid * ew, b)], raw.at[pl.ds(0, b)])
        pltpu.sync_copy(dst_h.at[pl.ds(wid * ew, b)], raw.at[pl.ds(b, b)])
        fill_idx(0)
        fire_gather(0, 0)
        pltpu.async_copy(ex_h.at[pl.ds(0, b)], alphab.at[pl.ds(0, b)], sem_e)

        if nchunks > 1:
            def body(blk, carry):
                slot1 = lax.rem(blk + 1, 2)
                fire_idx(blk + 1, slot1)
                for c in range(nchunks):
                    par = c % 2
                    wait_gather(par)
                    if c == 0:
                        wait_ex()
                    if c + 1 < nchunks:
                        fire_gather(c + 1, (c + 1) % 2)
                    else:
                        wait_idx()
                        fill_idx(slot1)
                        fire_gather(0, 0)
                    compute(c, par, 0, c == 0)
                finish(blk, 0)
                return carry

            lax.fori_loop(0, nblk, body, 0)
            wait_ex()
            wait_gather(0)
        else:
            def half(blk, par, slot1):
                fire_idx(blk + 1, slot1)
                wait_gather(par)
                wait_ex()
                wait_idx()
                fill_idx(slot1)
                fire_gather(0, 1 - par)
                compute(0, par, par, True)
                finish(blk, par)

            def body(i, carry):
                half(2 * i, 0, 1)
                half(2 * i + 1, 1, 0)
                return carry

            lax.fori_loop(0, nblk // 2, body, 0)
            if nblk % 2:
                half(nblk - 1, 0, 1)
                tailpar = 1
            else:
                tailpar = 0
            wait_ex()
            wait_gather(tailpar)

    return alpha_kernel


def _make_passb(npass, col_split):
    if col_split:
        ew = _E // _NS
    else:
        ew = _E // (_NC * _NS)
    bb = 400
    sb = 80
    nsb = bb // sb
    nblk = ew // bb
    ngrp = bb // _L
    rpt = 640
    rstride = 624
    zr = 40
    nchunks_out = npass * _NC if col_split else _NC
    nden = _N if col_split else _NC * _N

    @functools.partial(
        pl.kernel,
        mesh=_mesh(),
        out_type=[
            jax.ShapeDtypeStruct((nchunks_out * _N, _CW), jnp.float32),
            jax.ShapeDtypeStruct((nden, _CW), jnp.float32),
        ],
        scratch_types=[
            pltpu.VMEM((bb,), jnp.int32),
            pltpu.VMEM((bb,), jnp.int32),
            pltpu.VMEM((nsb, sb), jnp.int32),
            pltpu.VMEM((bb,), jnp.float32),
            pltpu.VMEM((3 * sb, _CW), jnp.float32),
            pltpu.VMEM((zr, _CW), jnp.float32),
            pltpu.VMEM((rpt,), jnp.float32),
            pltpu.VMEM((_L, _CW), jnp.float32),
            pltpu.VMEM_SHARED((_N, _CW), jnp.float32),
            pltpu.VMEM_SHARED((_N,), jnp.float32),
            pltpu.SemaphoreType.DMA,
        ],
        compiler_params=pltpu.CompilerParams(needs_layout_passes=False),
    )
    def passb_kernel(src_h, dst_h, ex_h, xls_h, out_h, den_h,
                     sidx, didx, didx2, exb, xlb, zbuf, dden, dbb,
                     accsp, denslab, sem):
        core = lax.axis_index("c")
        s = lax.axis_index("s")
        rowbase = s * rstride
        zv = jnp.zeros((_L,), jnp.float32)

        def zrow(r, carry):
            for k in range(_CW // _L):
                zbuf[r, pl.ds(k * _L, _L)] = zv
            return carry

        lax.fori_loop(0, zr, zrow, 0)

        def zden(i, carry):
            dden[pl.ds(i * _L, _L)] = zv
            return carry

        lax.fori_loop(0, rpt // _L, zden, 0)
        pltpu.sync_copy(dden, denslab.at[pl.ds(rowbase, rpt)])

        for p in range(npass):
            if col_split:
                chunk = core * npass + p
            else:
                chunk = core * 0
            rowoff = chunk * _N

            def zb(i, carry):
                pltpu.sync_copy(zbuf, accsp.at[pl.ds(rowbase + i * zr, zr)])
                return carry

            lax.fori_loop(0, rpt // zr, zb, 0)
            plsc.subcore_barrier()

            def blk_body(blk, carry, rowoff=rowoff, p=p):
                if col_split:
                    base = s * ew + blk * bb
                else:
                    base = core * (_E // _NC) + s * ew + blk * bb
                d1 = pltpu.async_copy(src_h.at[pl.ds(base, bb)], sidx, sem)
                d2 = pltpu.async_copy(dst_h.at[pl.ds(base, bb)], didx, sem)
                d3 = pltpu.async_copy(ex_h.at[pl.ds(base, bb)], exb, sem)
                d1.wait()
                d2.wait()
                d3.wait()

                def prep(g, c2, rowoff=rowoff):
                    sl = pl.ds(g * _L, _L)
                    dv = didx[sl]
                    j = g // (sb // _L)
                    didx2[j, pl.ds((g % (sb // _L)) * _L, _L)] = dv
                    if col_split:
                        sidx[sl] = sidx[sl] + rowoff
                    return c2

                for g in range(ngrp):
                    prep(g, 0)
                for j0, jn in ((0, 2), (2, 3)):
                    estart = j0 * sb
                    descs = []
                    for j in range(j0, j0 + jn):
                        jsl = pl.ds(j * sb, sb)
                        bsl = pl.ds((j - j0) * sb, sb)
                        descs.append(pltpu.async_copy(
                            xls_h.at[sidx.at[jsl]], xlb.at[bsl], sem))
                    for d in descs:
                        d.wait()

                    def ebody(g2, carry2, estart=estart):
                        exv = exb[pl.ds(estart + g2 * _L, _L)]
                        for lane in range(_L):
                            e2 = g2 * _L + lane
                            exs = exv[lane]
                            for k in range(_CW // _L):
                                sl = pl.ds(k * _L, _L)
                                xlb[e2, sl] = xlb[e2, sl] * exs
                        return carry2

                    lax.fori_loop(0, jn * sb // _L, ebody, 0)
                    for j in range(j0, j0 + jn):
                        jsl = pl.ds(j * sb, sb)
                        bsl = pl.ds((j - j0) * sb, sb)
                        pltpu.sync_copy(xlb.at[bsl], accsp.at[didx2.at[j]],
                                        add=True)
                        if p == 0:
                            pltpu.sync_copy(exb.at[jsl],
                                            denslab.at[didx2.at[j]], add=True)
                return carry

            lax.fori_loop(0, nblk, blk_body, 0)
            plsc.subcore_barrier()

            outrow = rowoff + rowbase if col_split else core * _N + rowbase
            pltpu.sync_copy(accsp.at[pl.ds(rowbase, rpt)],
                            out_h.at[pl.ds(outrow, rpt)])

            if p == 0:
                def den_stage():
                    pltpu.sync_copy(denslab.at[pl.ds(rowbase, rpt)], dden)
                    denrow = rowbase if col_split else core * _N + rowbase

                    def dbc(g, carry):
                        dv = dden[pl.ds(g * _L, _L)]
                        for lane in range(_L):
                            bc = zv + dv[lane]
                            for k in range(_CW // _L):
                                dbb[lane, pl.ds(k * _L, _L)] = bc
                        pltpu.sync_copy(
                            dbb, den_h.at[pl.ds(denrow + g * _L, _L)])
                        return carry

                    lax.fori_loop(0, rpt // _L, dbc, 0)

                if col_split:
                    @pl.when(core == 0)
                    def _():
                        den_stage()
                else:
                    den_stage()
            plsc.subcore_barrier()

    return passb_kernel


def _transform1(x, W_l, b_l, W_r, b_r):
    nb = 1000
    grid = (4, _N // nb)

    def body(x_ref, wl_ref, bl_ref, wr_ref, br_ref, xl_ref, xr_ref):
        c = pl.program_id(0)
        xb = x_ref[...]
        bl = bl_ref[pl.ds(c, 1), :]
        br = br_ref[pl.ds(c, 1), :]
        xl_ref[...] = jnp.dot(xb, wl_ref[...],
                              preferred_element_type=jnp.float32) + bl
        xr_ref[...] = jnp.dot(xb, wr_ref[...],
                              preferred_element_type=jnp.float32) + br

    out = pl.pallas_call(
        body,
        grid=grid,
        in_specs=[
            pl.BlockSpec((nb, 128), lambda c, i: (i, 0)),
            pl.BlockSpec((128, 128), lambda c, i: (0, c)),
            pl.BlockSpec((4, 128), lambda c, i: (0, 0)),
            pl.BlockSpec((128, 128), lambda c, i: (0, c)),
            pl.BlockSpec((4, 128), lambda c, i: (0, 0)),
        ],
        out_specs=[
            pl.BlockSpec((nb, 128), lambda c, i: (c * grid[1] + i, 0)),
            pl.BlockSpec((nb, 128), lambda c, i: (c * grid[1] + i, 0)),
        ],
        out_shape=[
            jax.ShapeDtypeStruct((4 * _N, 128), jnp.float32),
            jax.ShapeDtypeStruct((4 * _N, 128), jnp.float32),
        ],
    )(x, W_l, b_l, W_r, b_r)
    return out


def _transform2(o1, den1, bias1, W_l2, b_l2, W_r2, b_r2):
    nb = 1000
    grid = (_N // nb,)

    def body(o0, o1c, o2c, o3c, d_ref, b1_ref, wl_ref, bl_ref, wr_ref, br_ref,
             xl_ref, xr_ref):
        den = d_ref[...] + _EPS
        acc_l = jnp.zeros((nb, 128), jnp.float32)
        acc_r = jnp.zeros((nb, 128), jnp.float32)
        for c, oc in enumerate((o0, o1c, o2c, o3c)):
            h = jnp.maximum(oc[...] / den + b1_ref[c:c + 1, :], 0.0)
            acc_l += jnp.dot(h, wl_ref[c * 128:(c + 1) * 128, :],
                             preferred_element_type=jnp.float32)
            acc_r += jnp.dot(h, wr_ref[c * 128:(c + 1) * 128, :],
                             preferred_element_type=jnp.float32)
        xl_ref[...] = acc_l + bl_ref[...]
        xr_ref[...] = acc_r + br_ref[...]

    g1 = grid[0]
    in_specs = [pl.BlockSpec((nb, 128), lambda i, c=c: (c * g1 + i, 0))
                for c in range(4)]
    in_specs += [
        pl.BlockSpec((nb, 128), lambda i: (i, 0)),
        pl.BlockSpec((4, 128), lambda i: (0, 0)),
        pl.BlockSpec((512, 128), lambda i: (0, 0)),
        pl.BlockSpec((1, 128), lambda i: (0, 0)),
        pl.BlockSpec((512, 128), lambda i: (0, 0)),
        pl.BlockSpec((1, 128), lambda i: (0, 0)),
    ]
    return pl.pallas_call(
        body,
        grid=grid,
        in_specs=in_specs,
        out_specs=[
            pl.BlockSpec((nb, 128), lambda i: (i, 0)),
            pl.BlockSpec((nb, 128), lambda i: (i, 0)),
        ],
        out_shape=[
            jax.ShapeDtypeStruct((_N, 128), jnp.float32),
            jax.ShapeDtypeStruct((_N, 128), jnp.float32),
        ],
    )(o1, o1, o1, o1, den1, bias1, W_l2, b_l2, W_r2, b_r2)


def _final(o2, den2, bias2, W_cls, b_cls):
    nb = 1000
    grid = (_N // nb,)
    g1 = grid[0]

    def body(p0, p1, d0, d1, b2_ref, wc_ref, bc_ref, out_ref, acc):
        i = pl.program_id(0)
        num = p0[...] + p1[...]
        den = d0[...] + d1[...] + _EPS
        h = jnp.maximum(num / den + b2_ref[...], 0.0)
        psum = jnp.sum(h, axis=0, keepdims=True)

        @pl.when(i == 0)
        def _():
            acc[...] = psum

        @pl.when(i > 0)
        def _():
            acc[...] = acc[...] + psum

        @pl.when(i == g1 - 1)
        def _():
            pooled = acc[...] / float(_N)
            logits = jnp.dot(pooled, wc_ref[...],
                             preferred_element_type=jnp.float32) + bc_ref[...]
            out_ref[...] = jax.nn.sigmoid(logits)

    return pl.pallas_call(
        body,
        grid=grid,
        in_specs=[
            pl.BlockSpec((nb, 128), lambda i: (i, 0)),
            pl.BlockSpec((nb, 128), lambda i: (g1 + i, 0)),
            pl.BlockSpec((nb, 128), lambda i: (i, 0)),
            pl.BlockSpec((nb, 128), lambda i: (g1 + i, 0)),
            pl.BlockSpec((1, 128), lambda i: (0, 0)),
            pl.BlockSpec((128, 10), lambda i: (0, 0)),
            pl.BlockSpec((1, 10), lambda i: (0, 0)),
        ],
        out_specs=pl.BlockSpec((1, 10), lambda i: (0, 0)),
        out_shape=jax.ShapeDtypeStruct((1, 10), jnp.float32),
        scratch_shapes=[pltpu.VMEM((1, 128), jnp.float32)],
    )(o2, o2, den2, den2, bias2, W_cls, b_cls)


_alpha1 = _make_alpha(2, packed=True)
_alpha2 = _make_alpha(1)
_passb1 = _make_passb(2, True)
_passb2 = _make_passb(1, False)


def _pack_tc(xls, xrs):
    nb = 1000
    g1 = _N // nb

    def body(a0, b0, a1, b1, xo, ro):
        def pk(a, b):
            au = lax.bitcast_convert_type(
                a[...].astype(jnp.bfloat16), jnp.uint16).astype(jnp.uint32)
            bu = lax.bitcast_convert_type(
                b[...].astype(jnp.bfloat16), jnp.uint16).astype(jnp.uint32)
            return lax.bitcast_convert_type(au | (bu << 16), jnp.float32)

        xo[...] = pk(a0, b0)
        ro[...] = pk(a1, b1)

    return pl.pallas_call(
        body,
        grid=(2, g1),
        in_specs=[
            pl.BlockSpec((nb, 128), lambda q, i: (q * 2 * g1 + i, 0)),
            pl.BlockSpec((nb, 128), lambda q, i: (q * 2 * g1 + g1 + i, 0)),
            pl.BlockSpec((nb, 128), lambda q, i: (q * 2 * g1 + i, 0)),
            pl.BlockSpec((nb, 128), lambda q, i: (q * 2 * g1 + g1 + i, 0)),
        ],
        out_specs=[
            pl.BlockSpec((nb, 128), lambda q, i: (q * g1 + i, 0)),
            pl.BlockSpec((nb, 128), lambda q, i: (q * g1 + i, 0)),
        ],
        out_shape=[
            jax.ShapeDtypeStruct((2 * _N, 128), jnp.float32),
            jax.ShapeDtypeStruct((2 * _N, 128), jnp.float32),
        ],
    )(xls, xls, xrs, xrs)


def _pack_att(att):
    r = att.reshape(4, 128)
    lo = lax.bitcast_convert_type(
        jnp.stack([r[0], r[2]]).astype(jnp.bfloat16),
        jnp.uint16).astype(jnp.uint32)
    hi = lax.bitcast_convert_type(
        jnp.stack([r[1], r[3]]).astype(jnp.bfloat16),
        jnp.uint16).astype(jnp.uint32)
    return lax.bitcast_convert_type(lo | (hi << 16), jnp.float32).reshape(256)


def kernel(x, edge_index, W_l1, b_l1, W_r1, b_r1, att1, bias1,
           W_l2, b_l2, W_r2, b_r2, att2, bias2, W_cls, b_cls):
    src = edge_index[0].astype(jnp.int32)
    dst = edge_index[1].astype(jnp.int32)
    xls1, xrs1 = _transform1(x, W_l1, b_l1.reshape(4, 128),
                             W_r1, b_r1.reshape(4, 128))
    xls1p, xrs1p = _pack_tc(xls1, xrs1)
    ex1 = _alpha1(src, dst, _pack_att(att1), xls1p, xrs1p)
    o1, den1 = _passb1(src, dst, ex1, xls1)
    xl2, xr2 = _transform2(o1, den1, bias1.reshape(4, 128), W_l2,
                           b_l2.reshape(1, 128), W_r2, b_r2.reshape(1, 128))
    ex2 = _alpha2(src, dst, att2, xl2, xr2)
    o2, den2 = _passb2(src, dst, ex2, xl2)
    return _final(o2, den2, bias2.reshape(1, 128), W_cls, b_cls.reshape(1, 10))
